# baseline (speedup 1.0000x reference)
import functools

import jax
import jax.numpy as jnp
from jax.experimental import pallas as pl
from jax.experimental.pallas import tpu as pltpu

_EPS = 1e-06
_LANE = 128


def _loss_kernel(lbl_ref, x_hbm, off_ref, out_ref, buf0, buf1, sem0, sem1,
                 *, rows_per_core, chunk, n_chunks, inv_b):
    c = pl.program_id(0)
    base = c * rows_per_core

    def issue(buf, sem, ci):
        start = base + ci * chunk
        for j in range(chunk):
            row = start + j
            blk = lbl_ref[row] >> 7
            pltpu.make_async_copy(
                x_hbm.at[row, pl.ds(blk, 1)], buf.at[j], sem).start()

    def wait(buf, sem):
        pltpu.make_async_copy(buf, buf, sem).wait()

    lane = jax.lax.broadcasted_iota(jnp.int32, (chunk, 1, _LANE), 2)
    neg_log_eps = -jnp.log(jnp.float32(_EPS))

    def consume(buf, ci, acc):
        local = ci * chunk
        offs = off_ref[pl.ds(local, chunk)]
        vals = buf[...]
        logv = jnp.log(vals)
        per = jnp.where(logv != 0.0, -logv, neg_log_eps)
        contrib = jnp.where(lane == offs, per, 0.0)
        return acc + jnp.sum(contrib)

    half = n_chunks // 2
    issue(buf0, sem0, 0)

    def body(k, acc):
        ci0 = 2 * k
        issue(buf1, sem1, ci0 + 1)
        wait(buf0, sem0)
        acc = consume(buf0, ci0, acc)

        @pl.when(k + 1 < half)
        def _():
            issue(buf0, sem0, ci0 + 2)

        wait(buf1, sem1)
        acc = consume(buf1, ci0 + 1, acc)
        return acc

    acc = jax.lax.fori_loop(0, half, body, jnp.float32(0.0))
    out_ref[0, 0, 0] = acc * inv_b


@jax.jit
def kernel(output, label):
    B, C = output.shape
    assert C % _LANE == 0
    lbl = label.astype(jnp.int32)
    x3 = output.reshape(B, C // _LANE, _LANE)
    off3 = (lbl & (_LANE - 1)).reshape(B, 1, 1)

    n_cores = 2
    R = B // n_cores
    chunk = min(256, R)
    n_chunks = R // chunk
    assert n_chunks % 2 == 0 or n_chunks == 1

    if n_chunks == 1:
        chunk //= 2
        n_chunks = 2

    grid_spec = pltpu.PrefetchScalarGridSpec(
        num_scalar_prefetch=1,
        grid=(n_cores,),
        in_specs=[
            pl.BlockSpec(memory_space=pltpu.MemorySpace.HBM),
            pl.BlockSpec((R, 1, 1), lambda c, lbl_ref: (c, 0, 0)),
        ],
        out_specs=pl.BlockSpec((1, 1, 1), lambda c, lbl_ref: (c, 0, 0),
                               memory_space=pltpu.MemorySpace.SMEM),
        scratch_shapes=[
            pltpu.VMEM((chunk, 1, _LANE), jnp.float32),
            pltpu.VMEM((chunk, 1, _LANE), jnp.float32),
            pltpu.SemaphoreType.DMA,
            pltpu.SemaphoreType.DMA,
        ],
    )
    partial = pl.pallas_call(
        functools.partial(_loss_kernel, rows_per_core=R, chunk=chunk,
                          n_chunks=n_chunks, inv_b=1.0 / B),
        grid_spec=grid_spec,
        out_shape=jax.ShapeDtypeStruct((n_cores, 1, 1), jnp.float32),
        compiler_params=pltpu.CompilerParams(
            dimension_semantics=("parallel",),
            disable_bounds_checks=True,
        ),
        name="soft_max_loss_gather",
    )(lbl, x3, off3)
    return partial[0, 0, 0] + partial[1, 0, 0]
